# baseline (speedup 1.0000x reference)
import functools

import jax
import jax.numpy as jnp
from jax import lax
from jax.experimental import pallas as pl
from jax.experimental.pallas import tpu as pltpu
from jax.experimental.pallas import tpu_sc as plsc

_D = 128
_CS = 256
_NB = 4
_BLK = 8192
_SPLIT = 131072


def _sc_copy_tail(rows, split):
    info = plsc.get_sparse_core_info()
    nc, ns = info.num_cores, info.num_subcores
    nw = nc * ns
    tail = rows - split
    rpw = tail // nw
    n = rpw // _CS

    mesh = plsc.VectorSubcoreMesh(core_axis_name="c", subcore_axis_name="s")

    @functools.partial(
        pl.kernel,
        mesh=mesh,
        out_type=jax.ShapeDtypeStruct((rows, _D), jnp.float32),
        scratch_types=(
            [pltpu.VMEM_SHARED((ns, _CS, _D), jnp.float32) for _ in range(_NB)]
            + [pltpu.SemaphoreType.DMA for _ in range(2 * _NB)]
        ),
    )
    def sc_copy(src_hbm, dst_hbm, *scratch):
        shared = scratch[:_NB]
        sin = scratch[_NB : 2 * _NB]
        sout = scratch[2 * _NB :]
        cid = lax.axis_index("c")
        sid = lax.axis_index("s")
        wid = sid * nc + cid
        base = split + wid * rpw
        bufs = [shared[b].at[sid] for b in range(_NB)]

        in_copies = [None] * n
        out_copies = [None] * n
        for i in range(min(_NB, n)):
            in_copies[i] = pltpu.async_copy(
                src_hbm.at[pl.ds(base + i * _CS, _CS)], bufs[i % _NB], sin[i % _NB]
            )
        for i in range(n):
            b = i % _NB
            if i >= _NB:
                out_copies[i - _NB].wait()
                in_copies[i] = pltpu.async_copy(
                    src_hbm.at[pl.ds(base + i * _CS, _CS)], bufs[b], sin[b]
                )
            in_copies[i].wait()
            out_copies[i] = pltpu.async_copy(
                bufs[b], dst_hbm.at[pl.ds(base + i * _CS, _CS)], sout[b]
            )
        for i in range(max(0, n - _NB), n):
            out_copies[i].wait()

    return sc_copy


def _tc_copy_body(in_ref, out_ref):
    out_ref[...] = in_ref[...]


def _tc_copy_full(rows):
    spec = pl.BlockSpec((_BLK, _D), lambda i: (i, 0))
    return pl.pallas_call(
        _tc_copy_body,
        grid=(rows // _BLK,),
        out_shape=jax.ShapeDtypeStruct((rows, _D), jnp.float32),
        in_specs=[spec],
        out_specs=spec,
    )


def _tc_fill_head_body(in_ref, vbuf_ref, out_ref):
    out_ref[...] = in_ref[...]


def _tc_fill_head(rows, split):
    spec = pl.BlockSpec((_BLK, _D), lambda i: (i, 0))
    return pl.pallas_call(
        _tc_fill_head_body,
        grid=(split // _BLK,),
        out_shape=jax.ShapeDtypeStruct((rows, _D), jnp.float32),
        in_specs=[spec, pl.BlockSpec(memory_space=pl.ANY)],
        out_specs=spec,
        input_output_aliases={1: 0},
    )


def kernel(key_states, value_states, layer_idx):
    shape = key_states.shape
    rows = shape[0] * shape[1] * shape[2]
    k2 = key_states.reshape(rows, _D)
    v2 = value_states.reshape(rows, _D)
    vbuf = _sc_copy_tail(rows, _SPLIT)(v2)
    ko = _tc_copy_full(rows)(k2)
    vo = _tc_fill_head(rows, _SPLIT)(v2, vbuf)
    return (ko.reshape(shape), vo.reshape(shape))


# speedup vs baseline: 1.0915x; 1.0915x over previous
import jax
import jax.numpy as jnp
from jax.experimental import pallas as pl

_D = 128
_BLK = 4096


def _copy_kernel(k_in, v_in, k_out, v_out):
    k_out[...] = k_in[...]
    v_out[...] = v_in[...]


def kernel(key_states, value_states, layer_idx):
    shape = key_states.shape
    rows = shape[0] * shape[1] * shape[2]
    k2 = key_states.reshape(rows, _D)
    v2 = value_states.reshape(rows, _D)

    spec = pl.BlockSpec((_BLK, _D), lambda i: (i, 0))
    out_shape = (
        jax.ShapeDtypeStruct((rows, _D), key_states.dtype),
        jax.ShapeDtypeStruct((rows, _D), value_states.dtype),
    )
    ko, vo = pl.pallas_call(
        _copy_kernel,
        grid=(rows // _BLK,),
        out_shape=out_shape,
        in_specs=[spec, spec],
        out_specs=[spec, spec],
    )(k2, v2)
    return (ko.reshape(shape), vo.reshape(shape))


# speedup vs baseline: 1.1108x; 1.0177x over previous
import jax
import jax.numpy as jnp
from jax.experimental import pallas as pl

_D = 128
_BLK = 8192


def _copy_kernel(k_in, v_in, k_out, v_out):
    k_out[...] = k_in[...]
    v_out[...] = v_in[...]


def kernel(key_states, value_states, layer_idx):
    shape = key_states.shape
    rows = shape[0] * shape[1] * shape[2]
    k2 = key_states.reshape(rows, _D)
    v2 = value_states.reshape(rows, _D)

    spec = pl.BlockSpec((_BLK, _D), lambda i: (i, 0))
    out_shape = (
        jax.ShapeDtypeStruct((rows, _D), key_states.dtype),
        jax.ShapeDtypeStruct((rows, _D), value_states.dtype),
    )
    ko, vo = pl.pallas_call(
        _copy_kernel,
        grid=(rows // _BLK,),
        out_shape=out_shape,
        in_specs=[spec, spec],
        out_specs=[spec, spec],
    )(k2, v2)
    return (ko.reshape(shape), vo.reshape(shape))
